# Initial kernel scaffold; baseline (speedup 1.0000x reference)
import jax
import jax.numpy as jnp
from jax import lax
from jax.experimental import pallas as pl
from jax.experimental.pallas import tpu as pltpu
from jax.experimental.pallas import tpu_sc as plsc

EMB = 64
GROUP = 128
NC, NS = 2, 16
NW = NC * NS
B = 4096 * 50
NGROUPS = B // GROUP
GPW = NGROUPS // NW


def _lookup_kernel(node_table, edge_table, node_idx, edge_idx,
                   node_out, edge_out, idxn_v, idxe_v,
                   rows0, rows1, sem0, sem1):
    wid = lax.axis_index("s") * NC + lax.axis_index("c")
    g0 = wid * GPW

    pltpu.sync_copy(node_idx.at[pl.ds(g0, GPW)], idxn_v)
    pltpu.sync_copy(edge_idx.at[pl.ds(g0, GPW)], idxe_v)

    rows = (rows0, rows1)
    sems = (sem0, sem1)

    def run_table(table, idx_v, out):
        for b in range(2):
            pltpu.async_copy(table.at[idx_v.at[b]], rows[b], sems[b])

        def body(i, carry):
            for b in range(2):
                g = i * 2 + b
                pltpu.make_async_copy(
                    table.at[idx_v.at[g]], rows[b], sems[b]).wait()
                pltpu.sync_copy(rows[b], out.at[pl.ds((g0 + g) * GROUP, GROUP)])
                pltpu.async_copy(table.at[idx_v.at[g + 2]], rows[b], sems[b])
            return carry

        lax.fori_loop(0, GPW // 2 - 1, body, 0)

        for b in range(2):
            g = GPW - 2 + b
            pltpu.make_async_copy(
                table.at[idx_v.at[g]], rows[b], sems[b]).wait()
            pltpu.sync_copy(rows[b], out.at[pl.ds((g0 + g) * GROUP, GROUP)])

    run_table(node_table, idxn_v, node_out)
    run_table(edge_table, idxe_v, edge_out)


def kernel(node_table, edge_table, node_inputs, edge_inputs):
    out_shape = node_inputs.shape + (EMB,)
    node_idx = node_inputs.reshape(NGROUPS, GROUP).astype(jnp.int32)
    edge_idx = edge_inputs.reshape(NGROUPS, GROUP).astype(jnp.int32)

    mesh = plsc.VectorSubcoreMesh(
        core_axis_name="c", subcore_axis_name="s",
        num_cores=NC, num_subcores=NS)

    f = pl.kernel(
        _lookup_kernel,
        out_type=(jax.ShapeDtypeStruct((B, EMB), jnp.float32),
                  jax.ShapeDtypeStruct((B, EMB), jnp.float32)),
        mesh=mesh,
        scratch_types=[
            pltpu.VMEM((GPW, GROUP), jnp.int32),
            pltpu.VMEM((GPW, GROUP), jnp.int32),
            pltpu.VMEM((GROUP, EMB), jnp.float32),
            pltpu.VMEM((GROUP, EMB), jnp.float32),
            pltpu.SemaphoreType.DMA,
            pltpu.SemaphoreType.DMA,
        ],
    )
    node_flat, edge_flat = f(node_table, edge_table, node_idx, edge_idx)
    return (node_flat.reshape(out_shape), edge_flat.reshape(out_shape))


# speedup vs baseline: 1.6439x; 1.6439x over previous
import jax
import jax.numpy as jnp
from jax import lax
from jax.experimental import pallas as pl
from jax.experimental.pallas import tpu as pltpu
from jax.experimental.pallas import tpu_sc as plsc

EMB = 64
GROUP = 128
NC, NS = 2, 16
NW = NC * NS
B = 4096 * 50
NGROUPS = B // GROUP
GPW = NGROUPS // NW


def _lookup_kernel(node_table, edge_table, node_idx, edge_idx,
                   node_out, edge_out, idxn_v, idxe_v,
                   rows0, rows1, sem0, sem1):
    wid = lax.axis_index("s") * NC + lax.axis_index("c")
    g0 = wid * GPW

    pltpu.sync_copy(node_idx.at[wid], idxn_v)
    pltpu.sync_copy(edge_idx.at[wid], idxe_v)

    rows = (rows0, rows1)
    sems = (sem0, sem1)

    def run_table(table, idx_v, out):
        for b in range(2):
            pltpu.async_copy(table.at[idx_v.at[b]], rows[b], sems[b])

        def body(i, carry):
            for b in range(2):
                g = i * 2 + b
                pltpu.make_async_copy(
                    table.at[idx_v.at[g]], rows[b], sems[b]).wait()
                pltpu.sync_copy(rows[b], out.at[pl.ds((g0 + g) * GROUP, GROUP)])
                pltpu.async_copy(table.at[idx_v.at[g + 2]], rows[b], sems[b])
            return carry

        lax.fori_loop(0, GPW // 2 - 1, body, 0)

        for b in range(2):
            g = GPW - 2 + b
            pltpu.make_async_copy(
                table.at[idx_v.at[g]], rows[b], sems[b]).wait()
            pltpu.sync_copy(rows[b], out.at[pl.ds((g0 + g) * GROUP, GROUP)])

    run_table(node_table, idxn_v, node_out)
    run_table(edge_table, idxe_v, edge_out)


def kernel(node_table, edge_table, node_inputs, edge_inputs):
    out_shape = node_inputs.shape + (EMB,)
    node_idx = node_inputs.reshape(NW, GPW, GROUP).astype(jnp.int32)
    edge_idx = edge_inputs.reshape(NW, GPW, GROUP).astype(jnp.int32)

    mesh = plsc.VectorSubcoreMesh(
        core_axis_name="c", subcore_axis_name="s",
        num_cores=NC, num_subcores=NS)

    f = pl.kernel(
        _lookup_kernel,
        out_type=(jax.ShapeDtypeStruct((B, EMB), jnp.float32),
                  jax.ShapeDtypeStruct((B, EMB), jnp.float32)),
        mesh=mesh,
        compiler_params=pltpu.CompilerParams(use_tc_tiling_on_sc=False),
        scratch_types=[
            pltpu.VMEM((GPW, GROUP), jnp.int32),
            pltpu.VMEM((GPW, GROUP), jnp.int32),
            pltpu.VMEM((GROUP, EMB), jnp.float32),
            pltpu.VMEM((GROUP, EMB), jnp.float32),
            pltpu.SemaphoreType.DMA,
            pltpu.SemaphoreType.DMA,
        ],
    )
    node_flat, edge_flat = f(node_table, edge_table, node_idx, edge_idx)
    return (node_flat.reshape(out_shape), edge_flat.reshape(out_shape))


# speedup vs baseline: 1.6657x; 1.0132x over previous
import jax
import jax.numpy as jnp
from jax import lax
from jax.experimental import pallas as pl
from jax.experimental.pallas import tpu as pltpu
from jax.experimental.pallas import tpu_sc as plsc

EMB = 64
GROUP = 128
NC, NS = 2, 16
NW = NC * NS
B = 4096 * 50
NGROUPS = B // GROUP
GPW = NGROUPS // NW


NB = 8
K = 4
NR = (GPW - 2 * K) // NB
REM = (GPW - K) - (K + NR * NB)


def _lookup_kernel(node_table, edge_table, node_idx, edge_idx,
                   node_out, edge_out, idxn_v, idxe_v, rows, gsem, ssem):
    wid = lax.axis_index("s") * NC + lax.axis_index("c")
    g0 = wid * GPW

    pltpu.sync_copy(node_idx.at[wid], idxn_v)
    pltpu.sync_copy(edge_idx.at[wid], idxe_v)

    def run_table(table, idx_v, out):
        def fire_gather(g, slot):
            pltpu.async_copy(table.at[idx_v.at[g]], rows.at[slot], gsem.at[slot])

        def wait_gather(g, slot):
            pltpu.make_async_copy(
                table.at[idx_v.at[g]], rows.at[slot], gsem.at[slot]).wait()

        def fire_scatter(g, slot):
            pltpu.async_copy(
                rows.at[slot], out.at[pl.ds((g0 + g) * GROUP, GROUP)],
                ssem.at[slot])

        def wait_scatter(g, slot):
            pltpu.make_async_copy(
                rows.at[slot], out.at[pl.ds((g0 + g) * GROUP, GROUP)],
                ssem.at[slot]).wait()

        for g in range(K):
            fire_gather(g, g % NB)
        for g in range(K):
            fire_gather(g + K, (g + K) % NB)
            wait_gather(g, g % NB)
            fire_scatter(g, g % NB)
        def body(r, carry):
            for b in range(NB):
                g = K + r * NB + b
                slot_f = (2 * K + b) % NB
                wait_scatter(g + K - NB, slot_f)
                fire_gather(g + K, slot_f)
                wait_gather(g, (K + b) % NB)
                fire_scatter(g, (K + b) % NB)
            return carry

        lax.fori_loop(0, NR, body, 0)
        for g in range(K + NR * NB, GPW - K):
            slot_f = (g + K) % NB
            wait_scatter(g + K - NB, slot_f)
            fire_gather(g + K, slot_f)
            wait_gather(g, g % NB)
            fire_scatter(g, g % NB)
        for g in range(GPW - K, GPW):
            wait_gather(g, g % NB)
            fire_scatter(g, g % NB)
        for g in range(GPW - NB, GPW):
            wait_scatter(g, g % NB)

    run_table(node_table, idxn_v, node_out)
    run_table(edge_table, idxe_v, edge_out)


def kernel(node_table, edge_table, node_inputs, edge_inputs):
    out_shape = node_inputs.shape + (EMB,)
    node_idx = node_inputs.reshape(NW, GPW, GROUP).astype(jnp.int32)
    edge_idx = edge_inputs.reshape(NW, GPW, GROUP).astype(jnp.int32)

    mesh = plsc.VectorSubcoreMesh(
        core_axis_name="c", subcore_axis_name="s",
        num_cores=NC, num_subcores=NS)

    f = pl.kernel(
        _lookup_kernel,
        out_type=(jax.ShapeDtypeStruct((B, EMB), jnp.float32),
                  jax.ShapeDtypeStruct((B, EMB), jnp.float32)),
        mesh=mesh,
        compiler_params=pltpu.CompilerParams(use_tc_tiling_on_sc=False),
        scratch_types=[
            pltpu.VMEM((GPW, GROUP), jnp.int32),
            pltpu.VMEM((GPW, GROUP), jnp.int32),
            pltpu.VMEM((NB, GROUP, EMB), jnp.float32),
            pltpu.SemaphoreType.DMA((NB,)),
            pltpu.SemaphoreType.DMA((NB,)),
        ],
    )
    node_flat, edge_flat = f(node_table, edge_table, node_idx, edge_idx)
    return (node_flat.reshape(out_shape), edge_flat.reshape(out_shape))
